# Initial kernel scaffold; baseline (speedup 1.0000x reference)
import jax
import jax.numpy as jnp
from jax.experimental import pallas as pl


def kernel(query_id, cat_a, cat_b, cat_c, cat_d, numericals, vec_emb, query_table, ct_a, ct_b, ct_c, ct_d, num_W1, num_b1, num_W2, num_b2, vec_W, vec_b, merge_W1, merge_b1, merge_W2, merge_b2):
    raise NotImplementedError("write your pallas kernel here")


# speedup vs baseline: 1.1134x; 1.1134x over previous
import functools

import jax
import jax.numpy as jnp
from jax import lax
from jax.experimental import pallas as pl
from jax.experimental.pallas import tpu as pltpu
from jax.experimental.pallas import tpu_sc as plsc

B = 16384
TD = 32
NNUM = 8
VD = 128
NREP = 3
QED = 32
NFEAT = 9

NUM_TABLES = 5


def _gather_kernel(qt, ca_t, cb_t, cc_t, cd_t,
                   qid, ca, cb, cc, cd,
                   out_q, out_a, out_b, out_c, out_d,
                   i0, i1, i2, i3, i4, r0, r1, r2, r3, r4, sem):
  nc = 2
  b_per_w = B // 32
  wid = lax.axis_index("s") * nc + lax.axis_index("c")
  base = wid * b_per_w

  tables = (qt, ca_t, cb_t, cc_t, cd_t)
  idxs = (qid, ca, cb, cc, cd)
  outs = (out_q, out_a, out_b, out_c, out_d)
  idx_v = (i0, i1, i2, i3, i4)
  rows_v = (r0, r1, r2, r3, r4)

  for f in range(NUM_TABLES):
    pltpu.sync_copy(idxs[f].at[pl.ds(base, b_per_w)], idx_v[f])
  copies = []
  for f in range(NUM_TABLES):
    copies.append(
        pltpu.async_copy(tables[f].at[idx_v[f]], rows_v[f], sem))
  for c in copies:
    c.wait()
  for f in range(NUM_TABLES):
    pltpu.sync_copy(rows_v[f], outs[f].at[pl.ds(base, b_per_w)])


def _sc_gather(query_table, ct_a, ct_b, ct_c, ct_d, qid, ca, cb, cc, cd):
  b_per_w = B // 32
  mesh = plsc.VectorSubcoreMesh(core_axis_name="c", subcore_axis_name="s")
  out_t = tuple(
      jax.ShapeDtypeStruct((B, TD), jnp.float32) for _ in range(NUM_TABLES))
  fn = pl.kernel(
      _gather_kernel,
      out_type=out_t,
      mesh=mesh,
      scratch_types=(
          [pltpu.VMEM((b_per_w,), jnp.int32) for _ in range(NUM_TABLES)]
          + [pltpu.VMEM((b_per_w, TD), jnp.float32)
             for _ in range(NUM_TABLES)]
          + [pltpu.SemaphoreType.DMA]),
      compiler_params=pltpu.CompilerParams(use_tc_tiling_on_sc=False),
  )
  return fn(query_table, ct_a, ct_b, ct_c, ct_d, qid, ca, cb, cc, cd)


def _mlp_kernel(ea, eb, ec, ed, eq, num, vec,
                nw1, nb1, nw2, nb2, vw, vb, mw1, mb1, mw2, mb2,
                out):
  h = jnp.maximum(
      jnp.dot(num[...], nw1[...], preferred_element_type=jnp.float32)
      + nb1[...], 0.0)
  h = jnp.dot(h, nw2[...], preferred_element_type=jnp.float32) + nb2[...]
  v = jnp.dot(vec[...], vw[...], preferred_element_type=jnp.float32) + vb[...]
  feat = jnp.concatenate(
      [ea[...], eb[...], ec[...], ed[...], eq[...], h, v], axis=1)
  g = jnp.maximum(
      jnp.dot(feat, mw1[...], preferred_element_type=jnp.float32) + mb1[...],
      0.0)
  out[...] = (
      jnp.dot(g, mw2[...], preferred_element_type=jnp.float32) + mb2[...])


def _tc_mlp(emb_a, emb_b, emb_c, emb_d, emb_q, numericals, vec_emb,
            num_W1, num_b1, num_W2, num_b2, vec_W, vec_b,
            merge_W1, merge_b1, merge_W2, merge_b2):
  BB = 2048
  grid = (B // BB,)

  def batch_spec(width):
    return pl.BlockSpec((BB, width), lambda i: (i, 0))

  def full_spec(shape):
    return pl.BlockSpec(shape, lambda i: tuple(0 for _ in shape))

  return pl.pallas_call(
      _mlp_kernel,
      grid=grid,
      in_specs=[
          batch_spec(TD), batch_spec(TD), batch_spec(TD), batch_spec(TD),
          batch_spec(TD), batch_spec(NNUM), batch_spec(VD),
          full_spec(num_W1.shape), full_spec(num_b1.shape),
          full_spec(num_W2.shape), full_spec(num_b2.shape),
          full_spec(vec_W.shape), full_spec(vec_b.shape),
          full_spec(merge_W1.shape), full_spec(merge_b1.shape),
          full_spec(merge_W2.shape), full_spec(merge_b2.shape),
      ],
      out_specs=batch_spec(QED),
      out_shape=jax.ShapeDtypeStruct((B, QED), jnp.float32),
  )(emb_a, emb_b, emb_c, emb_d, emb_q, numericals, vec_emb,
    num_W1, num_b1, num_W2, num_b2, vec_W, vec_b,
    merge_W1, merge_b1, merge_W2, merge_b2)


def kernel(query_id, cat_a, cat_b, cat_c, cat_d, numericals, vec_emb,
           query_table, ct_a, ct_b, ct_c, ct_d,
           num_W1, num_b1, num_W2, num_b2,
           vec_W, vec_b,
           merge_W1, merge_b1, merge_W2, merge_b2):
  qid = query_id.astype(jnp.int32)
  ca = cat_a.astype(jnp.int32)
  cb = cat_b.astype(jnp.int32)
  cc = cat_c.astype(jnp.int32)
  cd = cat_d.astype(jnp.int32)

  eq, ea, eb, ec, ed = _sc_gather(
      query_table, ct_a, ct_b, ct_c, ct_d, qid, ca, cb, cc, cd)

  return _tc_mlp(
      ea, eb, ec, ed, eq, numericals, vec_emb,
      num_W1, num_b1.reshape(1, -1), num_W2, num_b2.reshape(1, -1),
      vec_W, vec_b.reshape(1, -1),
      merge_W1, merge_b1.reshape(1, -1), merge_W2, merge_b2.reshape(1, -1))
